# baseline (speedup 1.0000x reference)
import jax
import jax.numpy as jnp
from jax.experimental import pallas as pl
from jax.experimental.pallas import tpu as pltpu

N = 64
OBS_DIM = 1024
LOWD = 512
K = 512
H = 8
A = 128
HID = 512


def _fused(obs_ref, w_obs_hbm, b_obs_ref, w1_hbm, b1c_ref, idxT_ref,
           w2T_ref, b2c_ref, idx0_ref, action_ref, value_ref,
           w1a_s, w1o_s, w_obs_s, sem_a, sem_o, sem_w):
    cp_a = pltpu.make_async_copy(w1_hbm.at[pl.ds(LOWD, H * A), :], w1a_s, sem_a)
    cp_o = pltpu.make_async_copy(w1_hbm.at[pl.ds(0, LOWD), :], w1o_s, sem_o)
    cp_w = pltpu.make_async_copy(w_obs_hbm, w_obs_s, sem_w)
    cp_a.start()
    cp_o.start()
    cp_w.start()
    cp_a.wait()
    iota_a = jax.lax.broadcasted_iota(jnp.int32, (A, K), 0)
    actT = jnp.zeros((HID, K), dtype=jnp.float32)
    for h in range(H):
        onehotT = (iota_a == idxT_ref[h:h + 1, :]).astype(jnp.float32)
        actT = actT + jax.lax.dot_general(
            w1a_s[h * A:(h + 1) * A, :], onehotT,
            (((0,), (0,)), ((), ())), preferred_element_type=jnp.float32)

    cp_w.wait()
    cp_o.wait()
    obs_lowd = jnp.dot(obs_ref[...], w_obs_s[...],
                       preferred_element_type=jnp.float32) + b_obs_ref[...]
    oT = jax.lax.dot_general(
        w1o_s[...], obs_lowd, (((0,), (1,)), ((), ())),
        preferred_element_type=jnp.float32) + b1c_ref[...]
    w2T = w2T_ref[...]
    slabs = []
    for b in range(N):
        tb = jnp.maximum(actT + oT[:, b:b + 1], 0.0)
        slabs.append(jax.lax.dot_general(
            w2T, tb, (((1,), (0,)), ((), ())),
            preferred_element_type=jnp.float32))
    out3 = jnp.stack(slabs, axis=1) + b2c_ref[...][:, :, None]
    vals = out3[:H]
    lg = out3[H:]
    m = jnp.max(lg, axis=0, keepdims=True)
    e = jnp.exp(lg - m)
    s = jnp.sum(e, axis=0)
    v = jnp.sum(vals * e, axis=0) / s
    vmax = jnp.max(v, axis=1, keepdims=True)
    iota_k = jax.lax.broadcasted_iota(jnp.int32, (N, K), 1)
    karg = jnp.min(jnp.where(v >= vmax, iota_k, K), axis=1, keepdims=True)
    aidx = jnp.sum(jnp.where(iota_k == karg, idx0_ref[...], 0),
                   axis=1, keepdims=True)
    iota_act = jax.lax.broadcasted_iota(jnp.int32, (N, A), 1)
    action_ref[...] = (iota_act == aidx).astype(jnp.float32)
    value_ref[...] = vmax


@jax.jit
def kernel(observations, action_indices, W_obs, b_obs, W1, b1, W2, b2):
    idx = action_indices.reshape(K, H)
    action, value = pl.pallas_call(
        _fused,
        in_specs=[
            pl.BlockSpec(memory_space=pltpu.MemorySpace.VMEM),
            pl.BlockSpec(memory_space=pltpu.MemorySpace.HBM),
            pl.BlockSpec(memory_space=pltpu.MemorySpace.VMEM),
            pl.BlockSpec(memory_space=pltpu.MemorySpace.HBM),
            pl.BlockSpec(memory_space=pltpu.MemorySpace.VMEM),
            pl.BlockSpec(memory_space=pltpu.MemorySpace.VMEM),
            pl.BlockSpec(memory_space=pltpu.MemorySpace.VMEM),
            pl.BlockSpec(memory_space=pltpu.MemorySpace.VMEM),
            pl.BlockSpec(memory_space=pltpu.MemorySpace.VMEM),
        ],
        out_shape=(
            jax.ShapeDtypeStruct((N, A), jnp.float32),
            jax.ShapeDtypeStruct((N, 1), jnp.float32),
        ),
        scratch_shapes=[
            pltpu.VMEM((H * A, HID), jnp.float32),
            pltpu.VMEM((LOWD, HID), jnp.float32),
            pltpu.VMEM((OBS_DIM, LOWD), jnp.float32),
            pltpu.SemaphoreType.DMA,
            pltpu.SemaphoreType.DMA,
            pltpu.SemaphoreType.DMA,
        ],
    )(observations, W_obs, b_obs.reshape(1, LOWD), W1, b1.reshape(HID, 1),
      idx.T, W2.T, b2.reshape(2 * H, 1), idx[:, 0].reshape(1, K))
    return (action, value.reshape(N))
